# baseline (speedup 1.0000x reference)
import jax
import jax.numpy as jnp
from jax.experimental import pallas as pl
from jax.experimental.pallas import tpu as pltpu

_CHUNK_ROWS = 512
_N_CHUNKS = 32
_N_BUFS = 24


def _copy_body(x_ref, o_ref, bufs, in_sems, out_sems):
    def in_cp(i):
        return pltpu.make_async_copy(
            x_ref.at[pl.ds(i * _CHUNK_ROWS, _CHUNK_ROWS)],
            bufs.at[i % _N_BUFS],
            in_sems.at[i],
        )

    def out_cp(i):
        return pltpu.make_async_copy(
            bufs.at[i % _N_BUFS],
            o_ref.at[pl.ds(i * _CHUNK_ROWS, _CHUNK_ROWS)],
            out_sems.at[i],
        )

    for j in range(_N_BUFS):
        in_cp(j).start()
    for i in range(_N_CHUNKS):
        in_cp(i).wait()
        out_cp(i).start()
        nxt = i + _N_BUFS
        if nxt < _N_CHUNKS:
            out_cp(i).wait()
            in_cp(nxt).start()
    for i in range(_N_CHUNKS):
        if i + _N_BUFS >= _N_CHUNKS:
            out_cp(i).wait()


def kernel(x):
    shape = x.shape
    x2 = x.reshape(-1, shape[-1])
    out = pl.pallas_call(
        _copy_body,
        out_shape=jax.ShapeDtypeStruct(x2.shape, x2.dtype),
        in_specs=[pl.BlockSpec(memory_space=pl.ANY)],
        out_specs=pl.BlockSpec(memory_space=pl.ANY),
        scratch_shapes=[
            pltpu.VMEM((_N_BUFS, _CHUNK_ROWS, x2.shape[1]), x2.dtype),
            pltpu.SemaphoreType.DMA((_N_CHUNKS,)),
            pltpu.SemaphoreType.DMA((_N_CHUNKS,)),
        ],
    )(x2)
    return out.reshape(shape)


# speedup vs baseline: 1.0096x; 1.0096x over previous
import jax
import jax.numpy as jnp
from jax.experimental import pallas as pl
from jax.experimental.pallas import tpu as pltpu

_CHUNK_ROWS = 2048
_N_CHUNKS = 8
_N_BUFS = 6


def _copy_body(x_ref, o_ref, bufs, in_sems, out_sems):
    def in_cp(i):
        return pltpu.make_async_copy(
            x_ref.at[pl.ds(i * _CHUNK_ROWS, _CHUNK_ROWS)],
            bufs.at[i % _N_BUFS],
            in_sems.at[i],
        )

    def out_cp(i):
        return pltpu.make_async_copy(
            bufs.at[i % _N_BUFS],
            o_ref.at[pl.ds(i * _CHUNK_ROWS, _CHUNK_ROWS)],
            out_sems.at[i],
        )

    for j in range(_N_BUFS):
        in_cp(j).start()
    for i in range(_N_CHUNKS):
        in_cp(i).wait()
        out_cp(i).start()
        nxt = i + _N_BUFS
        if nxt < _N_CHUNKS:
            out_cp(i).wait()
            in_cp(nxt).start()
    for i in range(_N_CHUNKS):
        if i + _N_BUFS >= _N_CHUNKS:
            out_cp(i).wait()


def kernel(x):
    shape = x.shape
    x2 = x.reshape(-1, shape[-1])
    out = pl.pallas_call(
        _copy_body,
        out_shape=jax.ShapeDtypeStruct(x2.shape, x2.dtype),
        in_specs=[pl.BlockSpec(memory_space=pl.ANY)],
        out_specs=pl.BlockSpec(memory_space=pl.ANY),
        scratch_shapes=[
            pltpu.VMEM((_N_BUFS, _CHUNK_ROWS, x2.shape[1]), x2.dtype),
            pltpu.SemaphoreType.DMA((_N_CHUNKS,)),
            pltpu.SemaphoreType.DMA((_N_CHUNKS,)),
        ],
    )(x2)
    return out.reshape(shape)
